# baseline (speedup 1.0000x reference)
import functools

import jax
import jax.numpy as jnp
from jax import lax
from jax.experimental import pallas as pl
from jax.experimental.pallas import tpu as pltpu
from jax.experimental.pallas import tpu_sc as plsc

BATCH = 16384
EMB_DIM = 64
NUM_CORES = 2
NUM_SUBCORES = 16
LANES = 16
NUM_WORKERS = NUM_CORES * NUM_SUBCORES
BPW = BATCH // NUM_WORKERS

_mesh = plsc.VectorSubcoreMesh(
    core_axis_name="c", subcore_axis_name="s",
    num_cores=NUM_CORES, num_subcores=NUM_SUBCORES)


@functools.partial(
    pl.kernel,
    out_type=jax.ShapeDtypeStruct((BATCH,), jnp.float32),
    mesh=_mesh,
    compiler_params=pltpu.CompilerParams(
        use_tc_tiling_on_sc=False, needs_layout_passes=False),
    scratch_types=[
        pltpu.VMEM((2 * BPW,), jnp.int32),
        pltpu.VMEM((BPW,), jnp.int32),
        pltpu.VMEM((BPW,), jnp.int32),
        pltpu.VMEM((BPW, EMB_DIM), jnp.float32),
        pltpu.VMEM((BPW, EMB_DIM), jnp.float32),
        pltpu.VMEM((BPW,), jnp.float32),
        pltpu.SemaphoreType.DMA,
        pltpu.SemaphoreType.DMA,
    ],
)
def _mf_sc(x_hbm, user_hbm, item_hbm, out_hbm,
           xv, uidx, iidx, urows, irows, outv, sem_u, sem_i):
    wid = lax.axis_index("s") * NUM_CORES + lax.axis_index("c")
    base = wid * BPW

    pltpu.sync_copy(x_hbm.at[pl.ds(2 * base, 2 * BPW)], xv)

    def deint(g, carry):
        lanes = 2 * (g * LANES + lax.iota(jnp.int32, LANES))
        uidx[pl.ds(g * LANES, LANES)] = plsc.load_gather(xv, [lanes])
        iidx[pl.ds(g * LANES, LANES)] = plsc.load_gather(xv, [lanes + 1])
        return carry
    lax.fori_loop(0, BPW // LANES, deint, 0)

    cp_u = pltpu.async_copy(user_hbm.at[uidx], urows, sem_u)
    cp_i = pltpu.async_copy(item_hbm.at[iidx], irows, sem_i)
    cp_u.wait()
    cp_i.wait()

    def dot(g, carry):
        rows = g * LANES + lax.iota(jnp.int32, LANES)
        acc = jnp.zeros((LANES,), jnp.float32)
        for d in range(EMB_DIM):
            col = jnp.full((LANES,), d, jnp.int32)
            acc = acc + (plsc.load_gather(urows, [rows, col])
                         * plsc.load_gather(irows, [rows, col]))
        outv[pl.ds(g * LANES, LANES)] = acc
        return carry
    lax.fori_loop(0, BPW // LANES, dot, 0)

    pltpu.sync_copy(outv, out_hbm.at[pl.ds(base, BPW)])


@jax.jit
def kernel(x, user_table, item_table):
    out = _mf_sc(x.reshape(-1).astype(jnp.int32), user_table, item_table)
    return out[:, None]


# speedup vs baseline: 2.4830x; 2.4830x over previous
import functools

import jax
import jax.numpy as jnp
from jax import lax
from jax.experimental import pallas as pl
from jax.experimental.pallas import tpu as pltpu
from jax.experimental.pallas import tpu_sc as plsc

BATCH = 16384
VOCAB_SIZE = 1000000
EMB_DIM = 64
NUM_CORES = 2
NUM_SUBCORES = 16
LANES = 16
NUM_WORKERS = NUM_CORES * NUM_SUBCORES
BPW = BATCH // NUM_WORKERS

WIN = 512
MAIN_VOCAB = (VOCAB_SIZE // WIN) * WIN
NUM_WINDOWS = MAIN_VOCAB // WIN + 1
NUM_SLOTS = (NUM_WINDOWS + NUM_WORKERS - 1) // NUM_WORKERS
XCHUNK = 2048
RING = 32

_mesh = plsc.VectorSubcoreMesh(
    core_axis_name="c", subcore_axis_name="s",
    num_cores=NUM_CORES, num_subcores=NUM_SUBCORES)

_CP_TILED = pltpu.CompilerParams(
    use_tc_tiling_on_sc=True, needs_layout_passes=False)
_CP_LINEAR = pltpu.CompilerParams(
    use_tc_tiling_on_sc=False, needs_layout_passes=False)


@functools.partial(
    pl.kernel,
    out_type=(jax.ShapeDtypeStruct((BATCH * EMB_DIM,), jnp.float32),
              jax.ShapeDtypeStruct((BATCH * EMB_DIM,), jnp.float32)),
    mesh=_mesh,
    compiler_params=_CP_TILED,
    scratch_types=[
        pltpu.VMEM((XCHUNK,), jnp.int32),
        pltpu.VMEM((BATCH,), jnp.int32),
        pltpu.VMEM((BATCH,), jnp.int32),
        pltpu.VMEM((EMB_DIM, WIN), jnp.float32),
        pltpu.VMEM((EMB_DIM, WIN), jnp.float32),
        pltpu.VMEM((LANES,), jnp.int32),
        pltpu.VMEM((RING, EMB_DIM), jnp.float32),
        pltpu.VMEM((EMB_DIM,), jnp.float32),
        pltpu.SemaphoreType.DMA,
        pltpu.SemaphoreType.DMA,
    ],
)
def _mf_gather(x_hbm, uT_hbm, iT_hbm, utail_hbm, itail_hbm,
               ustage_hbm, istage_hbm,
               xv, ulist, ilist, uwin, iwin, svc, ring, drain,
               sem_win, sem_row):
    wid = lax.axis_index("s") * NUM_CORES + lax.axis_index("c")
    lane = lax.iota(jnp.int32, LANES)

    def build_list(dst_ref, x_base):
        def chunk(j, n):
            pltpu.sync_copy(x_hbm.at[pl.ds(x_base + j * XCHUNK, XCHUNK)], xv)
            def blk(g, nn):
                r = xv[pl.ds(g * LANES, LANES)]
                v = r >> 9
                m = (v & (NUM_WORKERS - 1)) == wid
                b = j * XCHUNK + g * LANES + lane
                pack = ((v >> 5) << 23) | ((r & (WIN - 1)) << 14) | b
                plsc.store_compressed(dst_ref.at[pl.ds(nn, LANES)], pack,
                                      mask=m)
                return nn + plsc.all_reduce_population_count(m)[0]
            return lax.fori_loop(0, XCHUNK // LANES, blk, n)
        return lax.fori_loop(0, BATCH // XCHUNK, chunk, 0)

    nu = build_list(ulist, 0)
    ni = build_list(ilist, BATCH)

    def serve(win_ref, list_ref, n, stage_ref, slot, hc0):
        nblk = (n + LANES - 1) // LANES
        def blk(t, hc):
            pk = list_ref[pl.ds(t * LANES, LANES)]
            valid = ((pk >> 23) == slot) & ((t * LANES + lane) < n)
            cnt = plsc.all_reduce_population_count(valid)[0]
            plsc.store_compressed(svc.at[pl.ds(0, LANES)], pk, mask=valid)
            sv = svc[pl.ds(0, LANES)]
            def each(i, hc2):
                pv = jnp.max(jnp.where(lane == i, sv, -1))
                lp = (pv >> 14) & (WIN - 1)
                b = pv & (BATCH - 1)
                lsp = jnp.full((LANES,), lp, jnp.int32)
                slotr = hc2 & (RING - 1)
                @pl.when(hc2 >= RING)
                def _():
                    pltpu.make_async_copy(
                        utail_hbm.at[0, pl.ds(0, EMB_DIM)], drain,
                        sem_row).wait()
                for q in range(EMB_DIM // LANES):
                    ring[slotr, pl.ds(q * LANES, LANES)] = plsc.load_gather(
                        win_ref, [lane + q * LANES, lsp])
                pltpu.async_copy(ring.at[slotr],
                                 stage_ref.at[pl.ds(b * EMB_DIM, EMB_DIM)],
                                 sem_row)
                return hc2 + 1
            return lax.fori_loop(0, cnt, each, hc)
        return lax.fori_loop(0, nblk, blk, hc0)

    def window(slot, hc):
        v = slot * NUM_WORKERS + wid
        @pl.when(v < NUM_WINDOWS - 1)
        def _():
            ds = []
            for a in range(EMB_DIM // 8):
                ds.append(pltpu.async_copy(
                    uT_hbm.at[pl.ds(8 * a, 8), pl.ds(v * WIN, WIN)],
                    uwin.at[pl.ds(8 * a, 8), :], sem_win))
                ds.append(pltpu.async_copy(
                    iT_hbm.at[pl.ds(8 * a, 8), pl.ds(v * WIN, WIN)],
                    iwin.at[pl.ds(8 * a, 8), :], sem_win))
            for d in ds:
                d.wait()
        @pl.when(v == NUM_WINDOWS - 1)
        def _():
            pltpu.sync_copy(utail_hbm, uwin.at[:, pl.ds(0, 128)])
            pltpu.sync_copy(itail_hbm, iwin.at[:, pl.ds(0, 128)])
        hc = serve(uwin, ulist, nu, ustage_hbm, slot, hc)
        hc = serve(iwin, ilist, ni, istage_hbm, slot, hc)
        return hc

    hc = lax.fori_loop(0, NUM_SLOTS, window, 0)

    def drain_one(i, c):
        pltpu.make_async_copy(
            utail_hbm.at[0, pl.ds(0, EMB_DIM)], drain, sem_row).wait()
        return c
    lax.fori_loop(0, jnp.minimum(hc, RING), drain_one, 0)


@functools.partial(
    pl.kernel,
    out_type=jax.ShapeDtypeStruct((BATCH,), jnp.float32),
    mesh=_mesh,
    compiler_params=_CP_LINEAR,
    scratch_types=[
        pltpu.VMEM((BPW, EMB_DIM), jnp.float32),
        pltpu.VMEM((BPW, EMB_DIM), jnp.float32),
        pltpu.VMEM((BPW,), jnp.float32),
    ],
)
def _mf_dot(ustage_hbm, istage_hbm, out_hbm, urows, irows, outv):
    wid = lax.axis_index("s") * NUM_CORES + lax.axis_index("c")
    base = wid * BPW
    pltpu.sync_copy(ustage_hbm.at[pl.ds(base, BPW), :], urows)
    pltpu.sync_copy(istage_hbm.at[pl.ds(base, BPW), :], irows)
    lane = lax.iota(jnp.int32, LANES)

    def dot(g, carry):
        rows = g * LANES + lane
        acc = jnp.zeros((LANES,), jnp.float32)
        for d in range(EMB_DIM):
            col = jnp.full((LANES,), d, jnp.int32)
            acc = acc + (plsc.load_gather(urows, [rows, col])
                         * plsc.load_gather(irows, [rows, col]))
        outv[pl.ds(g * LANES, LANES)] = acc
        return carry
    lax.fori_loop(0, BPW // LANES, dot, 0)
    pltpu.sync_copy(outv, out_hbm.at[pl.ds(base, BPW)])


@jax.jit
def kernel(x, user_table, item_table):
    xf = x.astype(jnp.int32).T.reshape(-1)
    uT = user_table.T
    iT = item_table.T
    utail = jnp.pad(user_table[MAIN_VOCAB:].T.astype(jnp.float32),
                    ((0, 0), (0, 128 - (VOCAB_SIZE - MAIN_VOCAB))))
    itail = jnp.pad(item_table[MAIN_VOCAB:].T.astype(jnp.float32),
                    ((0, 0), (0, 128 - (VOCAB_SIZE - MAIN_VOCAB))))
    ustage, istage = _mf_gather(xf, uT, iT, utail, itail)
    out = _mf_dot(ustage.reshape(BATCH, EMB_DIM),
                  istage.reshape(BATCH, EMB_DIM))
    return out[:, None]


# speedup vs baseline: 3.9174x; 1.5777x over previous
import functools

import jax
import jax.numpy as jnp
from jax import lax
from jax.experimental import pallas as pl
from jax.experimental.pallas import tpu as pltpu
from jax.experimental.pallas import tpu_sc as plsc

BATCH = 16384
VOCAB_SIZE = 1000000
EMB_DIM = 64
NUM_CORES = 2
NUM_SUBCORES = 16
LANES = 16
NUM_WORKERS = NUM_CORES * NUM_SUBCORES
BPW = BATCH // NUM_WORKERS

WIN = 512
MAIN_VOCAB = (VOCAB_SIZE // WIN) * WIN
NUM_WINDOWS = MAIN_VOCAB // WIN + 1
NUM_SLOTS = (NUM_WINDOWS + NUM_WORKERS - 1) // NUM_WORKERS
XCHUNK = 2048
RING = 32

_mesh = plsc.VectorSubcoreMesh(
    core_axis_name="c", subcore_axis_name="s",
    num_cores=NUM_CORES, num_subcores=NUM_SUBCORES)

_CP_TILED = pltpu.CompilerParams(
    use_tc_tiling_on_sc=True, needs_layout_passes=False)
_CP_LINEAR = pltpu.CompilerParams(
    use_tc_tiling_on_sc=False, needs_layout_passes=False)


@functools.partial(
    pl.kernel,
    out_type=(jax.ShapeDtypeStruct((BATCH * EMB_DIM,), jnp.float32),
              jax.ShapeDtypeStruct((BATCH * EMB_DIM,), jnp.float32)),
    mesh=_mesh,
    compiler_params=_CP_TILED,
    scratch_types=[
        pltpu.VMEM((2, XCHUNK), jnp.int32),
        pltpu.VMEM((BATCH + 3 * LANES,), jnp.int32),
        pltpu.VMEM((BATCH + 3 * LANES,), jnp.int32),
        pltpu.VMEM((NUM_SLOTS, LANES), jnp.int32),
        pltpu.VMEM((NUM_SLOTS, LANES), jnp.int32),
        pltpu.VMEM((EMB_DIM, WIN), jnp.float32),
        pltpu.VMEM((EMB_DIM, WIN), jnp.float32),
        pltpu.VMEM((LANES,), jnp.int32),
        pltpu.VMEM((RING, EMB_DIM), jnp.float32),
        pltpu.VMEM((EMB_DIM,), jnp.float32),
        pltpu.SemaphoreType.DMA,
        pltpu.SemaphoreType.DMA,
        pltpu.SemaphoreType.DMA,
        pltpu.SemaphoreType.DMA,
    ],
)
def _mf_gather(x_hbm, uT_hbm, iT_hbm, utail_hbm, itail_hbm,
               ustage_hbm, istage_hbm,
               xv, ulist, ilist, uocc, iocc, uwin, iwin, svc, ring, drain,
               sem_x, sem_uw, sem_iw, sem_row):
    wid = lax.axis_index("s") * NUM_CORES + lax.axis_index("c")
    lane = lax.iota(jnp.int32, LANES)
    ones = jnp.ones((LANES,), jnp.int32)

    def build_list(dst_ref, occ_ref, x_base):
        def zero(t, c):
            occ_ref[t, pl.ds(0, LANES)] = jnp.zeros((LANES,), jnp.int32)
            return c
        lax.fori_loop(0, NUM_SLOTS, zero, 0)
        pltpu.async_copy(x_hbm.at[pl.ds(x_base, XCHUNK)], xv.at[0], sem_x)
        def chunk(j, n):
            pltpu.make_async_copy(x_hbm.at[pl.ds(0, XCHUNK)],
                                  xv.at[0], sem_x).wait()
            @pl.when(j < BATCH // XCHUNK - 1)
            def _():
                pltpu.async_copy(
                    x_hbm.at[pl.ds(x_base + (j + 1) * XCHUNK, XCHUNK)],
                    xv.at[(j + 1) & 1], sem_x)
            def blk(g, nn):
                r = xv[j & 1, pl.ds(g * LANES, LANES)]
                v = r >> 9
                m = (v & (NUM_WORKERS - 1)) == wid
                b = j * XCHUNK + g * LANES + lane
                pack = ((v >> 5) << 23) | ((r & (WIN - 1)) << 14) | b
                plsc.store_compressed(dst_ref.at[pl.ds(nn, LANES)], pack,
                                      mask=m)
                plsc.addupdate_scatter(
                    occ_ref, [v >> 5, (r >> 7) & 3], ones, mask=m)
                return nn + plsc.all_reduce_population_count(m)[0]
            return lax.fori_loop(0, XCHUNK // LANES, blk, n)
        return lax.fori_loop(0, BATCH // XCHUNK, chunk, 0)

    nu = build_list(ulist, uocc, 0)
    ni = build_list(ilist, iocc, BATCH)

    def serve(win_ref, list_ref, n, stage_ref, slot, hc0):
        def half(pk, valid, cnt, hc):
            def hit_block():
                plsc.store_compressed(svc.at[pl.ds(0, LANES)], pk,
                                      mask=valid)
                sv = svc[pl.ds(0, LANES)]
                def each(i, hc2):
                    pv = jnp.max(jnp.where(lane == i, sv, -1))
                    lp = (pv >> 14) & (WIN - 1)
                    b = pv & (BATCH - 1)
                    lsp = jnp.full((LANES,), lp, jnp.int32)
                    slotr = hc2 & (RING - 1)
                    @pl.when(hc2 >= RING)
                    def _():
                        pltpu.make_async_copy(
                            utail_hbm.at[0, pl.ds(0, EMB_DIM)], drain,
                            sem_row).wait()
                    for q in range(EMB_DIM // LANES):
                        ring[slotr, pl.ds(q * LANES, LANES)] = (
                            plsc.load_gather(win_ref,
                                             [lane + q * LANES, lsp]))
                    pltpu.async_copy(
                        ring.at[slotr],
                        stage_ref.at[pl.ds(b * EMB_DIM, EMB_DIM)],
                        sem_row)
                    return hc2 + 1
                return lax.fori_loop(0, cnt, each, hc)
            return lax.cond(cnt > 0, hit_block, lambda: hc)

        UNROLL = 4
        nblku = (n + UNROLL * LANES - 1) // (UNROLL * LANES)
        def blk(t, hc):
            pks, valids, cnts = [], [], []
            for u in range(UNROLL):
                pk = list_ref[pl.ds((UNROLL * t + u) * LANES, LANES)]
                valid = (((pk >> 23) == slot)
                         & (((UNROLL * t + u) * LANES + lane) < n))
                pks.append(pk)
                valids.append(valid)
                cnts.append(plsc.all_reduce_population_count(valid)[0])
            for u in range(UNROLL):
                hc = half(pks[u], valids[u], cnts[u], hc)
            return hc
        return lax.fori_loop(0, nblku, blk, hc0)

    def fetch(tbl_hbm, buf, occ_ref, v, slot):
        cnts = occ_ref[slot, pl.ds(0, LANES)]
        k = 0
        for t in range(WIN // 128):
            @pl.when(cnts[t] > 0)
            def _():
                pltpu.async_copy(
                    tbl_hbm.at[:, pl.ds(v * WIN + t * 128, 128)],
                    buf.at[:, pl.ds(t * 128, 128)],
                    sem_uw if buf is uwin else sem_iw)
            k = k + (cnts[t] > 0).astype(jnp.int32)
        return k

    def waitk(buf, k):
        sem = sem_uw if buf is uwin else sem_iw
        def w(i, c):
            pltpu.make_async_copy(
                uT_hbm.at[:, pl.ds(0, 128)], buf.at[:, pl.ds(0, 128)],
                sem).wait()
            return c
        lax.fori_loop(0, k, w, 0)

    ku0 = fetch(uT_hbm, uwin, uocc, wid, 0)

    def window(slot, carry):
        hc, ku = carry
        v = slot * NUM_WORKERS + wid
        ki = fetch(iT_hbm, iwin, iocc, v, slot)
        waitk(uwin, ku)
        hc = serve(uwin, ulist, nu, ustage_hbm, slot, hc)
        ku_next = lax.cond(
            slot < NUM_SLOTS - 2,
            lambda: fetch(uT_hbm, uwin, uocc, v + NUM_WORKERS, slot + 1),
            lambda: 0)
        waitk(iwin, ki)
        hc = serve(iwin, ilist, ni, istage_hbm, slot, hc)
        return (hc, ku_next)

    hc, _ = lax.fori_loop(0, NUM_SLOTS - 1, window, (0, ku0))

    @pl.when(wid == 0)
    def _():
        pltpu.sync_copy(uT_hbm.at[:, pl.ds((NUM_WINDOWS - 2) * WIN, WIN)],
                        uwin)
        pltpu.sync_copy(iT_hbm.at[:, pl.ds((NUM_WINDOWS - 2) * WIN, WIN)],
                        iwin)
    @pl.when(wid == 1)
    def _():
        pltpu.sync_copy(utail_hbm, uwin.at[:, pl.ds(0, 128)])
        pltpu.sync_copy(itail_hbm, iwin.at[:, pl.ds(0, 128)])

    def last_slot(hc):
        hc = serve(uwin, ulist, nu, ustage_hbm, NUM_SLOTS - 1, hc)
        hc = serve(iwin, ilist, ni, istage_hbm, NUM_SLOTS - 1, hc)
        return hc
    hc = lax.cond(wid <= 1, last_slot, lambda h: h, hc)

    def drain_one(i, c):
        pltpu.make_async_copy(
            utail_hbm.at[0, pl.ds(0, EMB_DIM)], drain, sem_row).wait()
        return c
    lax.fori_loop(0, jnp.minimum(hc, RING), drain_one, 0)


@functools.partial(
    pl.kernel,
    out_type=jax.ShapeDtypeStruct((BATCH,), jnp.float32),
    mesh=_mesh,
    compiler_params=_CP_LINEAR,
    scratch_types=[
        pltpu.VMEM((BPW, EMB_DIM), jnp.float32),
        pltpu.VMEM((BPW, EMB_DIM), jnp.float32),
        pltpu.VMEM((BPW,), jnp.float32),
    ],
)
def _mf_dot(ustage_hbm, istage_hbm, out_hbm, urows, irows, outv):
    wid = lax.axis_index("s") * NUM_CORES + lax.axis_index("c")
    base = wid * BPW
    pltpu.sync_copy(ustage_hbm.at[pl.ds(base, BPW), :], urows)
    pltpu.sync_copy(istage_hbm.at[pl.ds(base, BPW), :], irows)
    lane = lax.iota(jnp.int32, LANES)

    def dot(g, carry):
        rows = g * LANES + lane
        acc = jnp.zeros((LANES,), jnp.float32)
        for d in range(EMB_DIM):
            col = jnp.full((LANES,), d, jnp.int32)
            acc = acc + (plsc.load_gather(urows, [rows, col])
                         * plsc.load_gather(irows, [rows, col]))
        outv[pl.ds(g * LANES, LANES)] = acc
        return carry
    lax.fori_loop(0, BPW // LANES, dot, 0)
    pltpu.sync_copy(outv, out_hbm.at[pl.ds(base, BPW)])


@jax.jit
def kernel(x, user_table, item_table):
    xf = x.astype(jnp.int32).T.reshape(-1)
    uT = user_table.T
    iT = item_table.T
    utail = jnp.pad(user_table[MAIN_VOCAB:].T.astype(jnp.float32),
                    ((0, 0), (0, 128 - (VOCAB_SIZE - MAIN_VOCAB))))
    itail = jnp.pad(item_table[MAIN_VOCAB:].T.astype(jnp.float32),
                    ((0, 0), (0, 128 - (VOCAB_SIZE - MAIN_VOCAB))))
    ustage, istage = _mf_gather(xf, uT, iT, utail, itail)
    out = _mf_dot(ustage.reshape(BATCH, EMB_DIM),
                  istage.reshape(BATCH, EMB_DIM))
    return out[:, None]
